# baseline (speedup 1.0000x reference)
import functools

import jax
import jax.numpy as jnp
from jax import lax
from jax.experimental import pallas as pl
from jax.experimental.pallas import tpu as pltpu
from jax.experimental.pallas import tpu_sc as plsc

_VOCAB = 1000000
_DIM = 64
_OUT_DIM = 5
_SEQ = 200
_BATCH = 4096

_NC = 2
_NS = 16
_NW = _NC * _NS
_BPW = _BATCH // _NW
_LANES = 16
_CH = _DIM // _LANES


def _sc_pool_body(text_hbm, emb_hbm, out_hbm, idx_v, rows_a, rows_b,
                  pooled_v, sem_a, sem_b):
    wid = lax.axis_index("s") * _NC + lax.axis_index("c")
    base = wid * _BPW

    pltpu.sync_copy(text_hbm.at[:, pl.ds(base, _BPW)], idx_v)

    zero = jnp.zeros((_LANES,), jnp.float32)

    def zbody(i, carry):
        for c in range(_CH):
            pooled_v[i, pl.ds(c * _LANES, _LANES)] = zero
        return carry

    lax.fori_loop(0, _BPW, zbody, 0, unroll=2)

    def fire(s, buf, sem):
        pltpu.async_copy(emb_hbm.at[idx_v.at[s]], buf, sem)

    def drain(buf, sem):
        pltpu.make_async_copy(emb_hbm.at[idx_v.at[0]], buf, sem).wait()

    def accum(rows):
        for b in range(_BPW):
            for c in range(_CH):
                sl = pl.ds(c * _LANES, _LANES)
                plsc.addupdate(pooled_v.at[b, sl], rows[b, sl])

    fire(0, rows_a, sem_a)
    fire(1, rows_b, sem_b)

    def pair_body(p, carry):
        s0 = 2 * p
        drain(rows_a, sem_a)
        accum(rows_a)

        @pl.when(s0 + 2 < _SEQ)
        def _():
            fire(s0 + 2, rows_a, sem_a)

        drain(rows_b, sem_b)
        accum(rows_b)

        @pl.when(s0 + 3 < _SEQ)
        def _():
            fire(s0 + 3, rows_b, sem_b)

        return carry

    lax.fori_loop(0, _SEQ // 2, pair_body, 0)

    pltpu.sync_copy(pooled_v, out_hbm.at[pl.ds(base, _BPW), :])


@functools.partial(
    pl.kernel,
    out_type=jax.ShapeDtypeStruct((_BATCH, _DIM), jnp.float32),
    mesh=plsc.VectorSubcoreMesh(core_axis_name="c", subcore_axis_name="s"),
    compiler_params=pltpu.CompilerParams(use_tc_tiling_on_sc=False),
    scratch_types=[
        pltpu.VMEM((_SEQ, _BPW), jnp.int32),
        pltpu.VMEM((_BPW, _DIM), jnp.float32),
        pltpu.VMEM((_BPW, _DIM), jnp.float32),
        pltpu.VMEM((_BPW, _DIM), jnp.float32),
        pltpu.SemaphoreType.DMA,
        pltpu.SemaphoreType.DMA,
    ],
)
def _sc_pool(text_hbm, emb_hbm, out_hbm, idx_v, rows_a, rows_b, pooled_v,
             sem_a, sem_b):
    _sc_pool_body(text_hbm, emb_hbm, out_hbm, idx_v, rows_a, rows_b,
                  pooled_v, sem_a, sem_b)


def _mm_body(p_ref, w_ref, b_ref, o_ref):
    acc = lax.dot_general(p_ref[...], w_ref[...],
                          (((1,), (1,)), ((), ())),
                          preferred_element_type=jnp.float32)
    o_ref[...] = acc * (1.0 / _SEQ) + b_ref[...]


def kernel(text, emb, W, b):
    sums = _sc_pool(text.astype(jnp.int32), emb)
    out = pl.pallas_call(
        _mm_body,
        out_shape=jax.ShapeDtypeStruct((_BATCH, _OUT_DIM), jnp.float32),
    )(sums, W, b.reshape(1, _OUT_DIM))
    return out


# speedup vs baseline: 1.6555x; 1.6555x over previous
import functools

import jax
import jax.numpy as jnp
from jax import lax
from jax.experimental import pallas as pl
from jax.experimental.pallas import tpu as pltpu
from jax.experimental.pallas import tpu_sc as plsc

_VOCAB = 1000000
_DIM = 64
_OUT_DIM = 5
_SEQ = 200
_BATCH = 4096

_NC = 2
_NS = 16
_NW = _NC * _NS
_BPW = _BATCH // _NW
_LANES = 16
_CH = _DIM // _LANES


def _sc_pool_body(text_hbm, emb_hbm, out_hbm, idx_v, rows_a, rows_b,
                  pooled_v, sem_a, sem_b):
    wid = lax.axis_index("s") * _NC + lax.axis_index("c")
    base = wid * _BPW

    pltpu.sync_copy(text_hbm.at[:, pl.ds(base, _BPW)], idx_v)

    zero = jnp.zeros((_LANES,), jnp.float32)

    def zbody(i, carry):
        for c in range(_CH):
            pooled_v[i, pl.ds(c * _LANES, _LANES)] = zero
        return carry

    lax.fori_loop(0, _BPW, zbody, 0, unroll=2)

    def fire_add(s, sem):
        pltpu.async_copy(emb_hbm.at[idx_v.at[s]], pooled_v, sem, add=True)

    def drain_one(sem):
        pltpu.make_async_copy(emb_hbm.at[idx_v.at[0]], pooled_v, sem).wait()

    for s in range(8):
        fire_add(s, sem_a)

    def body(p, carry):
        drain_one(sem_a)

        @pl.when(p + 8 < _SEQ)
        def _():
            fire_add(p + 8, sem_a)

        return carry

    lax.fori_loop(0, _SEQ, body, 0)

    pltpu.sync_copy(pooled_v, out_hbm.at[pl.ds(base, _BPW), :])


@functools.partial(
    pl.kernel,
    out_type=jax.ShapeDtypeStruct((_BATCH, _DIM), jnp.float32),
    mesh=plsc.VectorSubcoreMesh(core_axis_name="c", subcore_axis_name="s"),
    compiler_params=pltpu.CompilerParams(use_tc_tiling_on_sc=False),
    scratch_types=[
        pltpu.VMEM((_SEQ, _BPW), jnp.int32),
        pltpu.VMEM((_BPW, _DIM), jnp.float32),
        pltpu.VMEM((_BPW, _DIM), jnp.float32),
        pltpu.VMEM((_BPW, _DIM), jnp.float32),
        pltpu.SemaphoreType.DMA,
        pltpu.SemaphoreType.DMA,
    ],
)
def _sc_pool(text_hbm, emb_hbm, out_hbm, idx_v, rows_a, rows_b, pooled_v,
             sem_a, sem_b):
    _sc_pool_body(text_hbm, emb_hbm, out_hbm, idx_v, rows_a, rows_b,
                  pooled_v, sem_a, sem_b)


def _mm_body(p_ref, w_ref, b_ref, o_ref):
    acc = lax.dot_general(p_ref[...], w_ref[...],
                          (((1,), (1,)), ((), ())),
                          preferred_element_type=jnp.float32)
    o_ref[...] = acc * (1.0 / _SEQ) + b_ref[...]


def kernel(text, emb, W, b):
    sums = _sc_pool(text.astype(jnp.int32), emb)
    out = pl.pallas_call(
        _mm_body,
        out_shape=jax.ShapeDtypeStruct((_BATCH, _OUT_DIM), jnp.float32),
    )(sums, W, b.reshape(1, _OUT_DIM))
    return out
